# Initial kernel scaffold; baseline (speedup 1.0000x reference)
import jax
import jax.numpy as jnp
from jax.experimental import pallas as pl


def kernel(node_features, edge_index, Wg, bg, W1, b1, W2, b2):
    raise NotImplementedError("write your pallas kernel here")


# speedup vs baseline: 22.8962x; 22.8962x over previous
import functools

import jax
import jax.numpy as jnp
from jax import lax
from jax.experimental import pallas as pl
from jax.experimental.pallas import tpu as pltpu
from jax.experimental.pallas import tpu_sc as plsc

_B, _N, _F, _H, _E = 4, 10000, 128, 128, 160000
_NN = _B * _N
_NE = _B * _E

_DCH = 2000
_EPT_DEG = _NE // 32


@functools.cache
def _make_deg_kernel():
    return functools.partial(
        pl.kernel,
        out_type=jax.ShapeDtypeStruct((32, 1, _NN), jnp.float32),
        mesh=plsc.VectorSubcoreMesh(core_axis_name="c", subcore_axis_name="s"),
        compiler_params=pltpu.CompilerParams(needs_layout_passes=False),
        scratch_types=[
            pltpu.VMEM((_NN,), jnp.float32),
            pltpu.VMEM((_DCH,), jnp.int32),
        ],
    )(_deg_body)


def _deg_body(dst_hbm, hist_hbm, hist_v, idx_v):
    c = lax.axis_index("c")
    s = lax.axis_index("s")
    wid = s * 2 + c
    e0 = wid * _EPT_DEG
    boff = (e0 // _E) * _N

    def zero_body(i, carry):
        hist_v[pl.ds(i * 16, 16)] = jnp.zeros((16,), jnp.float32)
        return carry

    lax.fori_loop(0, _NN // 16, zero_body, 0)

    ones = jnp.ones((16,), jnp.float32)

    def chunk_body(k, carry):
        pltpu.sync_copy(dst_hbm.at[pl.ds(e0 + k * _DCH, _DCH)], idx_v)

        def inner(j, cin):
            iv = idx_v[pl.ds(j * 16, 16)] + boff
            plsc.addupdate_scatter(hist_v, [iv], ones)
            return cin

        lax.fori_loop(0, _DCH // 16, inner, 0)
        return carry

    lax.fori_loop(0, _EPT_DEG // _DCH, chunk_body, 0)
    pltpu.sync_copy(hist_v, hist_hbm.at[wid, 0])


_BR2 = 2000


def _y_body(x_ref, hist_ref, wg_ref, y_ref, dinv_ref):
    deg = jnp.sum(hist_ref[...], axis=1) + 1.0
    dinv = lax.rsqrt(deg)
    y = jnp.dot(x_ref[...], wg_ref[...], preferred_element_type=jnp.float32)
    y_ref[...] = y * dinv[:, None]
    dinv_ref[...] = dinv[:, None]


def _y_stage(x, hist_t, Wg):
    return pl.pallas_call(
        _y_body,
        grid=(_NN // _BR2,),
        in_specs=[
            pl.BlockSpec((_BR2, _F), lambda i: (i, 0)),
            pl.BlockSpec((_BR2, 32), lambda i: (i, 0)),
            pl.BlockSpec((_F, _F), lambda i: (0, 0)),
        ],
        out_specs=[
            pl.BlockSpec((_BR2, _F), lambda i: (i, 0)),
            pl.BlockSpec((_BR2, 1), lambda i: (i, 0)),
        ],
        out_shape=[
            jax.ShapeDtypeStruct((_NN, _F), jnp.float32),
            jax.ShapeDtypeStruct((_NN, 1), jnp.float32),
        ],
    )(x, hist_t, Wg)


_OC = 400
_T = 80
_NT = _OC // _T
_EPT = _E // 16
_RT0 = 640
_RT = 624


@functools.cache
def _make_agg_kernel():
    return functools.partial(
        pl.kernel,
        out_type=jax.ShapeDtypeStruct((_NN, _F), jnp.float32),
        mesh=plsc.VectorSubcoreMesh(core_axis_name="c", subcore_axis_name="s"),
        compiler_params=pltpu.CompilerParams(needs_layout_passes=False),
        scratch_types=[
            pltpu.VMEM_SHARED((_N, _F), jnp.float32),
            pltpu.VMEM((_OC,), jnp.int32),
            pltpu.VMEM((_OC,), jnp.int32),
            pltpu.VMEM((_NT, _T), jnp.int32),
            pltpu.VMEM((_NT, _T), jnp.int32),
            pltpu.VMEM((_T, _F), jnp.float32),
            pltpu.SemaphoreType.DMA,
        ],
    )(_agg_body)


def _agg_body(y_hbm, src_hbm, dstl_hbm, acc_hbm,
              acc_sh, sraw, draw, src2, dst2, rows, sem):
    c = lax.axis_index("c")
    t = lax.axis_index("s")
    r0 = jnp.where(t == 0, 0, _RT0 + (t - 1) * _RT)

    for cc in range(2):
        b = c * 2 + cc
        nbase = b * _N
        e_base = b * _E + t * _EPT

        def _seed(nq, nr):
            def qb(q, carry):
                rr = r0 + q * nr
                hop = rows.at[pl.ds(0, nr)]
                pltpu.sync_copy(y_hbm.at[pl.ds(nbase + rr, nr)], hop)
                pltpu.sync_copy(hop, acc_sh.at[pl.ds(rr, nr)])
                return carry
            lax.fori_loop(0, nq, qb, 0)

        @pl.when(t == 0)
        def _():
            _seed(8, _RT0 // 8)

        @pl.when(t != 0)
        def _():
            _seed(13, _RT // 13)

        plsc.subcore_barrier()

        def oc_body(k, carry):
            eoff = e_base + k * _OC
            pltpu.sync_copy(src_hbm.at[pl.ds(eoff, _OC)], sraw)
            pltpu.sync_copy(dstl_hbm.at[pl.ds(eoff, _OC)], draw)
            for j in range(_NT):
                for l in range(_T // 16):
                    sl = pl.ds(l * 16, 16)
                    fl = pl.ds(j * _T + l * 16, 16)
                    src2[j, sl] = sraw[fl]
                    dst2[j, sl] = draw[fl]
            for j in range(_NT):
                pltpu.async_copy(y_hbm.at[src2.at[j]], rows, sem).wait()
                pltpu.sync_copy(rows, acc_sh.at[dst2.at[j]], add=True)
            return carry

        lax.fori_loop(0, _EPT // _OC, oc_body, 0)
        plsc.subcore_barrier()

        def _flush(nq, nr):
            def qb(q, carry):
                rr = r0 + q * nr
                hop = rows.at[pl.ds(0, nr)]
                pltpu.sync_copy(acc_sh.at[pl.ds(rr, nr)], hop)
                pltpu.sync_copy(hop, acc_hbm.at[pl.ds(nbase + rr, nr)])
                return carry
            lax.fori_loop(0, nq, qb, 0)

        @pl.when(t == 0)
        def _():
            _flush(8, _RT0 // 8)

        @pl.when(t != 0)
        def _():
            _flush(13, _RT // 13)

        plsc.subcore_barrier()


_BRM = 1000


def _mlp_body(acc_ref, x_ref, dinv_ref, bg_ref, w1_ref, b1_ref,
              w2_ref, b2_ref, out_ref):
    b = pl.program_id(0)
    i = pl.program_id(1)
    h = acc_ref[...] * dinv_ref[...] + bg_ref[...]
    h = jnp.maximum(h, 0.0) + x_ref[...]
    h1 = jnp.dot(h, w1_ref[...], preferred_element_type=jnp.float32)
    h1 = h1 + b1_ref[...]
    h1 = jnp.where(h1 > 0, h1, 0.01 * h1)
    h2 = jnp.dot(h1, w2_ref[...], preferred_element_type=jnp.float32)
    h2 = h2 + b2_ref[...]
    h2 = jnp.where(h2 > 0, h2, 0.01 * h2)
    part = jnp.sum(h2, axis=0, keepdims=True)

    @pl.when(i == 0)
    def _():
        out_ref[pl.ds(b, 1), :] = part

    @pl.when(i != 0)
    def _():
        out_ref[pl.ds(b, 1), :] += part


def _mlp_stage(acc, x, dinv, bg, W1, b1, W2, b2):
    nb = _N // _BRM
    return pl.pallas_call(
        _mlp_body,
        grid=(_B, nb),
        in_specs=[
            pl.BlockSpec((_BRM, _F), lambda b, i: (b * nb + i, 0)),
            pl.BlockSpec((_BRM, _F), lambda b, i: (b * nb + i, 0)),
            pl.BlockSpec((_BRM, 1), lambda b, i: (b * nb + i, 0)),
            pl.BlockSpec((1, _F), lambda b, i: (0, 0)),
            pl.BlockSpec((_F, _H), lambda b, i: (0, 0)),
            pl.BlockSpec((1, _H), lambda b, i: (0, 0)),
            pl.BlockSpec((_H, _H), lambda b, i: (0, 0)),
            pl.BlockSpec((1, _H), lambda b, i: (0, 0)),
        ],
        out_specs=pl.BlockSpec((_B, _H), lambda b, i: (0, 0)),
        out_shape=jax.ShapeDtypeStruct((_B, _H), jnp.float32),
    )(acc, x, dinv, bg, W1, b1, W2, b2)


def kernel(node_features, edge_index, Wg, bg, W1, b1, W2, b2):
    x = node_features.reshape(_NN, _F)
    offs = (jnp.arange(_B, dtype=jnp.int32) * _N)[:, None]
    src_g = (edge_index[:, 0, :] + offs).reshape(_NE)
    dst_l = edge_index[:, 1, :].reshape(_NE)
    hist = _make_deg_kernel()(dst_l)
    hist_t = hist.reshape(32, _NN).T
    y, dinv = _y_stage(x, hist_t, Wg)
    acc = _make_agg_kernel()(y, src_g, dst_l)
    return _mlp_stage(acc, x, dinv, bg.reshape(1, _F), W1,
                      b1.reshape(1, _H), W2, b2.reshape(1, _H))


# speedup vs baseline: 33.9666x; 1.4835x over previous
import functools

import jax
import jax.numpy as jnp
from jax import lax
from jax.experimental import pallas as pl
from jax.experimental.pallas import tpu as pltpu
from jax.experimental.pallas import tpu_sc as plsc

_B, _N, _F, _H, _E = 4, 10000, 128, 128, 160000
_NN = _B * _N
_NE = _B * _E

_DCH = 2000
_EPT_DEG = _NE // 32


@functools.cache
def _make_deg_kernel():
    return functools.partial(
        pl.kernel,
        out_type=jax.ShapeDtypeStruct((32, 1, _NN), jnp.float32),
        mesh=plsc.VectorSubcoreMesh(core_axis_name="c", subcore_axis_name="s"),
        compiler_params=pltpu.CompilerParams(needs_layout_passes=False),
        scratch_types=[
            pltpu.VMEM((_NN,), jnp.float32),
            pltpu.VMEM((_DCH,), jnp.int32),
        ],
    )(_deg_body)


def _deg_body(dst_hbm, hist_hbm, hist_v, idx_v):
    c = lax.axis_index("c")
    s = lax.axis_index("s")
    wid = s * 2 + c
    e0 = wid * _EPT_DEG
    boff = (e0 // _E) * _N

    def zero_body(i, carry):
        hist_v[pl.ds(i * 16, 16)] = jnp.zeros((16,), jnp.float32)
        return carry

    lax.fori_loop(0, _NN // 16, zero_body, 0)

    ones = jnp.ones((16,), jnp.float32)

    def chunk_body(k, carry):
        pltpu.sync_copy(dst_hbm.at[pl.ds(e0 + k * _DCH, _DCH)], idx_v)

        def inner(j, cin):
            iv = idx_v[pl.ds(j * 16, 16)] + boff
            plsc.addupdate_scatter(hist_v, [iv], ones)
            return cin

        lax.fori_loop(0, _DCH // 16, inner, 0)
        return carry

    lax.fori_loop(0, _EPT_DEG // _DCH, chunk_body, 0)
    pltpu.sync_copy(hist_v, hist_hbm.at[wid, 0])


_BR2 = 2000


def _y_body(x_ref, hist_ref, wg_ref, y_ref, dinv_ref):
    deg = jnp.sum(hist_ref[...], axis=1) + 1.0
    dinv = lax.rsqrt(deg)
    y = jnp.dot(x_ref[...], wg_ref[...], preferred_element_type=jnp.float32)
    y_ref[...] = y * dinv[:, None]
    dinv_ref[...] = dinv[:, None]


def _y_stage(x, hist_t, Wg):
    return pl.pallas_call(
        _y_body,
        grid=(_NN // _BR2,),
        in_specs=[
            pl.BlockSpec((_BR2, _F), lambda i: (i, 0)),
            pl.BlockSpec((_BR2, 32), lambda i: (i, 0)),
            pl.BlockSpec((_F, _F), lambda i: (0, 0)),
        ],
        out_specs=[
            pl.BlockSpec((_BR2, _F), lambda i: (i, 0)),
            pl.BlockSpec((_BR2, 1), lambda i: (i, 0)),
        ],
        out_shape=[
            jax.ShapeDtypeStruct((_NN, _F), jnp.float32),
            jax.ShapeDtypeStruct((_NN, 1), jnp.float32),
        ],
    )(x, hist_t, Wg)


_OC = 2000
_T = 80
_NT = _OC // _T
_EPT = _E // 16
_RT0 = 640
_RT = 624


@functools.cache
def _make_agg_kernel():
    return functools.partial(
        pl.kernel,
        out_type=jax.ShapeDtypeStruct((_NN, _F), jnp.float32),
        mesh=plsc.VectorSubcoreMesh(core_axis_name="c", subcore_axis_name="s"),
        compiler_params=pltpu.CompilerParams(needs_layout_passes=False),
        scratch_types=[
            pltpu.VMEM_SHARED((_N, _F), jnp.float32),
            pltpu.VMEM((_OC,), jnp.int32),
            pltpu.VMEM((_OC,), jnp.int32),
            pltpu.VMEM((_NT, _T), jnp.int32),
            pltpu.VMEM((_NT, _T), jnp.int32),
            pltpu.VMEM((_T, _F), jnp.float32),
            pltpu.VMEM((_T, _F), jnp.float32),
            pltpu.SemaphoreType.DMA,
            pltpu.SemaphoreType.DMA,
        ],
    )(_agg_body)


def _agg_body(y_hbm, src_hbm, dstl_hbm, acc_hbm,
              acc_sh, sraw, draw, src2, dst2, rows_a, rows_b, sem_a, sem_b):
    c = lax.axis_index("c")
    t = lax.axis_index("s")
    r0 = jnp.where(t == 0, 0, _RT0 + (t - 1) * _RT)

    for cc in range(2):
        b = c * 2 + cc
        nbase = b * _N
        e_base = b * _E + t * _EPT

        def _seed(nq, nr):
            def qb(q, carry):
                rr = r0 + q * nr
                hop = rows_a.at[pl.ds(0, nr)]
                pltpu.sync_copy(y_hbm.at[pl.ds(nbase + rr, nr)], hop)
                pltpu.sync_copy(hop, acc_sh.at[pl.ds(rr, nr)])
                return carry
            lax.fori_loop(0, nq, qb, 0)

        @pl.when(t == 0)
        def _():
            _seed(8, _RT0 // 8)

        @pl.when(t != 0)
        def _():
            _seed(13, _RT // 13)

        plsc.subcore_barrier()

        def oc_body(k, carry):
            eoff = e_base + k * _OC
            pltpu.sync_copy(src_hbm.at[pl.ds(eoff, _OC)], sraw)
            pltpu.sync_copy(dstl_hbm.at[pl.ds(eoff, _OC)], draw)
            for j in range(_NT):
                for l in range(_T // 16):
                    sl = pl.ds(l * 16, 16)
                    fl = pl.ds(j * _T + l * 16, 16)
                    src2[j, sl] = sraw[fl]
                    dst2[j, sl] = draw[fl]
            bufs = (rows_a, rows_b)
            sems = (sem_a, sem_b)
            gets = [None] * _NT
            gets[0] = pltpu.async_copy(y_hbm.at[src2.at[0]], bufs[0], sems[0])
            for j in range(_NT):
                if j + 1 < _NT:
                    gets[j + 1] = pltpu.async_copy(
                        y_hbm.at[src2.at[j + 1]], bufs[(j + 1) % 2],
                        sems[(j + 1) % 2])
                gets[j].wait()
                pltpu.sync_copy(bufs[j % 2], acc_sh.at[dst2.at[j]], add=True)
            return carry

        lax.fori_loop(0, _EPT // _OC, oc_body, 0)
        plsc.subcore_barrier()

        def _flush(nq, nr):
            def qb(q, carry):
                rr = r0 + q * nr
                hop = rows_a.at[pl.ds(0, nr)]
                pltpu.sync_copy(acc_sh.at[pl.ds(rr, nr)], hop)
                pltpu.sync_copy(hop, acc_hbm.at[pl.ds(nbase + rr, nr)])
                return carry
            lax.fori_loop(0, nq, qb, 0)

        @pl.when(t == 0)
        def _():
            _flush(8, _RT0 // 8)

        @pl.when(t != 0)
        def _():
            _flush(13, _RT // 13)

        plsc.subcore_barrier()


_BRM = 1000


def _mlp_body(acc_ref, x_ref, dinv_ref, bg_ref, w1_ref, b1_ref,
              w2_ref, b2_ref, out_ref):
    b = pl.program_id(0)
    i = pl.program_id(1)
    h = acc_ref[...] * dinv_ref[...] + bg_ref[...]
    h = jnp.maximum(h, 0.0) + x_ref[...]
    h1 = jnp.dot(h, w1_ref[...], preferred_element_type=jnp.float32)
    h1 = h1 + b1_ref[...]
    h1 = jnp.where(h1 > 0, h1, 0.01 * h1)
    h2 = jnp.dot(h1, w2_ref[...], preferred_element_type=jnp.float32)
    h2 = h2 + b2_ref[...]
    h2 = jnp.where(h2 > 0, h2, 0.01 * h2)
    part = jnp.sum(h2, axis=0, keepdims=True)

    @pl.when(i == 0)
    def _():
        out_ref[pl.ds(b, 1), :] = part

    @pl.when(i != 0)
    def _():
        out_ref[pl.ds(b, 1), :] += part


def _mlp_stage(acc, x, dinv, bg, W1, b1, W2, b2):
    nb = _N // _BRM
    return pl.pallas_call(
        _mlp_body,
        grid=(_B, nb),
        in_specs=[
            pl.BlockSpec((_BRM, _F), lambda b, i: (b * nb + i, 0)),
            pl.BlockSpec((_BRM, _F), lambda b, i: (b * nb + i, 0)),
            pl.BlockSpec((_BRM, 1), lambda b, i: (b * nb + i, 0)),
            pl.BlockSpec((1, _F), lambda b, i: (0, 0)),
            pl.BlockSpec((_F, _H), lambda b, i: (0, 0)),
            pl.BlockSpec((1, _H), lambda b, i: (0, 0)),
            pl.BlockSpec((_H, _H), lambda b, i: (0, 0)),
            pl.BlockSpec((1, _H), lambda b, i: (0, 0)),
        ],
        out_specs=pl.BlockSpec((_B, _H), lambda b, i: (0, 0)),
        out_shape=jax.ShapeDtypeStruct((_B, _H), jnp.float32),
    )(acc, x, dinv, bg, W1, b1, W2, b2)


def kernel(node_features, edge_index, Wg, bg, W1, b1, W2, b2):
    x = node_features.reshape(_NN, _F)
    offs = (jnp.arange(_B, dtype=jnp.int32) * _N)[:, None]
    src_g = (edge_index[:, 0, :] + offs).reshape(_NE)
    dst_l = edge_index[:, 1, :].reshape(_NE)
    hist = _make_deg_kernel()(dst_l)
    hist_t = hist.reshape(32, _NN).T
    y, dinv = _y_stage(x, hist_t, Wg)
    acc = _make_agg_kernel()(y, src_g, dst_l)
    return _mlp_stage(acc, x, dinv, bg.reshape(1, _F), W1,
                      b1.reshape(1, _H), W2, b2.reshape(1, _H))


# speedup vs baseline: 33.9681x; 1.0000x over previous
import functools

import jax
import jax.numpy as jnp
from jax import lax
from jax.experimental import pallas as pl
from jax.experimental.pallas import tpu as pltpu
from jax.experimental.pallas import tpu_sc as plsc

_B, _N, _F, _H, _E = 4, 10000, 128, 128, 160000
_NN = _B * _N
_NE = _B * _E

_DCH = 2000
_EPT_DEG = _NE // 32


@functools.cache
def _make_deg_kernel():
    return functools.partial(
        pl.kernel,
        out_type=jax.ShapeDtypeStruct((32, 1, _NN), jnp.float32),
        mesh=plsc.VectorSubcoreMesh(core_axis_name="c", subcore_axis_name="s"),
        compiler_params=pltpu.CompilerParams(needs_layout_passes=False),
        scratch_types=[
            pltpu.VMEM((_NN,), jnp.float32),
            pltpu.VMEM((_DCH,), jnp.int32),
        ],
    )(_deg_body)


def _deg_body(dst_hbm, hist_hbm, hist_v, idx_v):
    c = lax.axis_index("c")
    s = lax.axis_index("s")
    wid = s * 2 + c
    e0 = wid * _EPT_DEG
    boff = (e0 // _E) * _N

    def zero_body(i, carry):
        hist_v[pl.ds(i * 16, 16)] = jnp.zeros((16,), jnp.float32)
        return carry

    lax.fori_loop(0, _NN // 16, zero_body, 0)

    ones = jnp.ones((16,), jnp.float32)

    def chunk_body(k, carry):
        pltpu.sync_copy(dst_hbm.at[pl.ds(e0 + k * _DCH, _DCH)], idx_v)

        def inner(j, cin):
            iv = idx_v[pl.ds(j * 16, 16)] + boff
            plsc.addupdate_scatter(hist_v, [iv], ones)
            return cin

        lax.fori_loop(0, _DCH // 16, inner, 0)
        return carry

    lax.fori_loop(0, _EPT_DEG // _DCH, chunk_body, 0)
    pltpu.sync_copy(hist_v, hist_hbm.at[wid, 0])


_BR2 = 2000


def _y_body(x_ref, hist_ref, wg_ref, y_ref, dinv_ref):
    deg = jnp.sum(hist_ref[...], axis=1) + 1.0
    dinv = lax.rsqrt(deg)
    y = jnp.dot(x_ref[...], wg_ref[...], preferred_element_type=jnp.float32)
    y_ref[...] = y * dinv[:, None]
    dinv_ref[...] = dinv[:, None]


def _y_stage(x, hist_t, Wg):
    return pl.pallas_call(
        _y_body,
        grid=(_NN // _BR2,),
        in_specs=[
            pl.BlockSpec((_BR2, _F), lambda i: (i, 0)),
            pl.BlockSpec((_BR2, 32), lambda i: (i, 0)),
            pl.BlockSpec((_F, _F), lambda i: (0, 0)),
        ],
        out_specs=[
            pl.BlockSpec((_BR2, _F), lambda i: (i, 0)),
            pl.BlockSpec((_BR2, 1), lambda i: (i, 0)),
        ],
        out_shape=[
            jax.ShapeDtypeStruct((_NN, _F), jnp.float32),
            jax.ShapeDtypeStruct((_NN, 1), jnp.float32),
        ],
    )(x, hist_t, Wg)


_OC = 2000
_T = 80
_NT = _OC // _T
_EPT = _E // 16
_RT0 = 640
_RT = 624


@functools.cache
def _make_agg_kernel():
    return functools.partial(
        pl.kernel,
        out_type=jax.ShapeDtypeStruct((_NN, _F), jnp.float32),
        mesh=plsc.VectorSubcoreMesh(core_axis_name="c", subcore_axis_name="s"),
        compiler_params=pltpu.CompilerParams(needs_layout_passes=False),
        scratch_types=[
            pltpu.VMEM_SHARED((_N, _F), jnp.float32),
            pltpu.VMEM((_OC,), jnp.int32),
            pltpu.VMEM((_OC,), jnp.int32),
            pltpu.VMEM((_NT, _T), jnp.int32),
            pltpu.VMEM((_NT, _T), jnp.int32),
            pltpu.VMEM((_T, _F), jnp.float32),
            pltpu.VMEM((_T, _F), jnp.float32),
            pltpu.SemaphoreType.DMA,
            pltpu.SemaphoreType.DMA,
            pltpu.SemaphoreType.DMA,
            pltpu.SemaphoreType.DMA,
        ],
    )(_agg_body)


def _agg_body(y_hbm, src_hbm, dstl_hbm, acc_hbm,
              acc_sh, sraw, draw, src2, dst2, rows_a, rows_b,
              sem_a, sem_b, ssem_a, ssem_b):
    c = lax.axis_index("c")
    t = lax.axis_index("s")
    r0 = jnp.where(t == 0, 0, _RT0 + (t - 1) * _RT)

    for cc in range(2):
        b = c * 2 + cc
        nbase = b * _N
        e_base = b * _E + t * _EPT

        def _seed(nq, nr):
            def qb(q, carry):
                rr = r0 + q * nr
                hop = rows_a.at[pl.ds(0, nr)]
                pltpu.sync_copy(y_hbm.at[pl.ds(nbase + rr, nr)], hop)
                pltpu.sync_copy(hop, acc_sh.at[pl.ds(rr, nr)])
                return carry
            lax.fori_loop(0, nq, qb, 0)

        @pl.when(t == 0)
        def _():
            _seed(8, _RT0 // 8)

        @pl.when(t != 0)
        def _():
            _seed(13, _RT // 13)

        plsc.subcore_barrier()

        def oc_body(k, carry):
            eoff = e_base + k * _OC
            pltpu.sync_copy(src_hbm.at[pl.ds(eoff, _OC)], sraw)
            pltpu.sync_copy(dstl_hbm.at[pl.ds(eoff, _OC)], draw)
            for j in range(_NT):
                for l in range(_T // 16):
                    sl = pl.ds(l * 16, 16)
                    fl = pl.ds(j * _T + l * 16, 16)
                    src2[j, sl] = sraw[fl]
                    dst2[j, sl] = draw[fl]
            bufs = (rows_a, rows_b)
            sems = (sem_a, sem_b)
            ssems = (ssem_a, ssem_b)
            gets = [None] * _NT
            puts = [None] * _NT
            gets[0] = pltpu.async_copy(y_hbm.at[src2.at[0]], bufs[0], sems[0])
            for j in range(_NT):
                if j >= 1:
                    puts[j - 1].wait()
                if j + 1 < _NT:
                    gets[j + 1] = pltpu.async_copy(
                        y_hbm.at[src2.at[j + 1]], bufs[(j + 1) % 2],
                        sems[(j + 1) % 2])
                gets[j].wait()
                puts[j] = pltpu.async_copy(
                    bufs[j % 2], acc_sh.at[dst2.at[j]], ssems[j % 2],
                    add=True)
            puts[_NT - 1].wait()
            return carry

        lax.fori_loop(0, _EPT // _OC, oc_body, 0)
        plsc.subcore_barrier()

        def _flush(nq, nr):
            def qb(q, carry):
                rr = r0 + q * nr
                hop = rows_a.at[pl.ds(0, nr)]
                pltpu.sync_copy(acc_sh.at[pl.ds(rr, nr)], hop)
                pltpu.sync_copy(hop, acc_hbm.at[pl.ds(nbase + rr, nr)])
                return carry
            lax.fori_loop(0, nq, qb, 0)

        @pl.when(t == 0)
        def _():
            _flush(8, _RT0 // 8)

        @pl.when(t != 0)
        def _():
            _flush(13, _RT // 13)

        plsc.subcore_barrier()


_BRM = 1000


def _mlp_body(acc_ref, x_ref, dinv_ref, bg_ref, w1_ref, b1_ref,
              w2_ref, b2_ref, out_ref):
    b = pl.program_id(0)
    i = pl.program_id(1)
    h = acc_ref[...] * dinv_ref[...] + bg_ref[...]
    h = jnp.maximum(h, 0.0) + x_ref[...]
    h1 = jnp.dot(h, w1_ref[...], preferred_element_type=jnp.float32)
    h1 = h1 + b1_ref[...]
    h1 = jnp.where(h1 > 0, h1, 0.01 * h1)
    h2 = jnp.dot(h1, w2_ref[...], preferred_element_type=jnp.float32)
    h2 = h2 + b2_ref[...]
    h2 = jnp.where(h2 > 0, h2, 0.01 * h2)
    part = jnp.sum(h2, axis=0, keepdims=True)

    @pl.when(i == 0)
    def _():
        out_ref[pl.ds(b, 1), :] = part

    @pl.when(i != 0)
    def _():
        out_ref[pl.ds(b, 1), :] += part


def _mlp_stage(acc, x, dinv, bg, W1, b1, W2, b2):
    nb = _N // _BRM
    return pl.pallas_call(
        _mlp_body,
        grid=(_B, nb),
        in_specs=[
            pl.BlockSpec((_BRM, _F), lambda b, i: (b * nb + i, 0)),
            pl.BlockSpec((_BRM, _F), lambda b, i: (b * nb + i, 0)),
            pl.BlockSpec((_BRM, 1), lambda b, i: (b * nb + i, 0)),
            pl.BlockSpec((1, _F), lambda b, i: (0, 0)),
            pl.BlockSpec((_F, _H), lambda b, i: (0, 0)),
            pl.BlockSpec((1, _H), lambda b, i: (0, 0)),
            pl.BlockSpec((_H, _H), lambda b, i: (0, 0)),
            pl.BlockSpec((1, _H), lambda b, i: (0, 0)),
        ],
        out_specs=pl.BlockSpec((_B, _H), lambda b, i: (0, 0)),
        out_shape=jax.ShapeDtypeStruct((_B, _H), jnp.float32),
    )(acc, x, dinv, bg, W1, b1, W2, b2)


def kernel(node_features, edge_index, Wg, bg, W1, b1, W2, b2):
    x = node_features.reshape(_NN, _F)
    offs = (jnp.arange(_B, dtype=jnp.int32) * _N)[:, None]
    src_g = (edge_index[:, 0, :] + offs).reshape(_NE)
    dst_l = edge_index[:, 1, :].reshape(_NE)
    hist = _make_deg_kernel()(dst_l)
    hist_t = hist.reshape(32, _NN).T
    y, dinv = _y_stage(x, hist_t, Wg)
    acc = _make_agg_kernel()(y, src_g, dst_l)
    return _mlp_stage(acc, x, dinv, bg.reshape(1, _F), W1,
                      b1.reshape(1, _H), W2, b2.reshape(1, _H))


# speedup vs baseline: 37.1260x; 1.0930x over previous
import functools

import jax
import jax.numpy as jnp
from jax import lax
from jax.experimental import pallas as pl
from jax.experimental.pallas import tpu as pltpu
from jax.experimental.pallas import tpu_sc as plsc

_B, _N, _F, _H, _E = 4, 10000, 128, 128, 160000
_NN = _B * _N
_NE = _B * _E

_DCH = 2000
_EPT_DEG = _NE // 32


@functools.cache
def _make_deg_kernel():
    return functools.partial(
        pl.kernel,
        out_type=jax.ShapeDtypeStruct((32, 1, _NN), jnp.float32),
        mesh=plsc.VectorSubcoreMesh(core_axis_name="c", subcore_axis_name="s"),
        compiler_params=pltpu.CompilerParams(needs_layout_passes=False),
        scratch_types=[
            pltpu.VMEM((_NN,), jnp.float32),
            pltpu.VMEM((_DCH,), jnp.int32),
        ],
    )(_deg_body)


def _deg_body(dst_hbm, hist_hbm, hist_v, idx_v):
    c = lax.axis_index("c")
    s = lax.axis_index("s")
    wid = s * 2 + c
    e0 = wid * _EPT_DEG
    boff = (e0 // _E) * _N

    def zero_body(i, carry):
        hist_v[pl.ds(i * 16, 16)] = jnp.zeros((16,), jnp.float32)
        return carry

    lax.fori_loop(0, _NN // 16, zero_body, 0)

    ones = jnp.ones((16,), jnp.float32)

    def chunk_body(k, carry):
        pltpu.sync_copy(dst_hbm.at[pl.ds(e0 + k * _DCH, _DCH)], idx_v)

        def inner(j, cin):
            for u in range(5):
                iv = idx_v[pl.ds(j * 80 + u * 16, 16)] + boff
                plsc.addupdate_scatter(hist_v, [iv], ones)
            return cin

        lax.fori_loop(0, _DCH // 80, inner, 0)
        return carry

    lax.fori_loop(0, _EPT_DEG // _DCH, chunk_body, 0)
    pltpu.sync_copy(hist_v, hist_hbm.at[wid, 0])


_BR2 = 2000


def _y_body(x_ref, hist_ref, wg_ref, y_ref, dinv_ref):
    deg = jnp.sum(hist_ref[...], axis=1) + 1.0
    dinv = lax.rsqrt(deg)
    y = jnp.dot(x_ref[...], wg_ref[...], preferred_element_type=jnp.float32)
    y_ref[...] = y * dinv[:, None]
    dinv_ref[...] = dinv[:, None]


def _y_stage(x, hist_t, Wg):
    return pl.pallas_call(
        _y_body,
        grid=(_NN // _BR2,),
        in_specs=[
            pl.BlockSpec((_BR2, _F), lambda i: (i, 0)),
            pl.BlockSpec((_BR2, 32), lambda i: (i, 0)),
            pl.BlockSpec((_F, _F), lambda i: (0, 0)),
        ],
        out_specs=[
            pl.BlockSpec((_BR2, _F), lambda i: (i, 0)),
            pl.BlockSpec((_BR2, 1), lambda i: (i, 0)),
        ],
        out_shape=[
            jax.ShapeDtypeStruct((_NN, _F), jnp.float32),
            jax.ShapeDtypeStruct((_NN, 1), jnp.float32),
        ],
    )(x, hist_t, Wg)


_OC = 2000
_T = 80
_NT = _OC // _T
_EPT = _E // 16
_RT0 = 640
_RT = 624


@functools.cache
def _make_agg_kernel():
    return functools.partial(
        pl.kernel,
        out_type=jax.ShapeDtypeStruct((_NN, _F), jnp.float32),
        mesh=plsc.VectorSubcoreMesh(core_axis_name="c", subcore_axis_name="s"),
        compiler_params=pltpu.CompilerParams(needs_layout_passes=False),
        scratch_types=[
            pltpu.VMEM_SHARED((_N, _F), jnp.float32),
            pltpu.VMEM((_OC,), jnp.int32),
            pltpu.VMEM((_OC,), jnp.int32),
            pltpu.VMEM((_NT, _T), jnp.int32),
            pltpu.VMEM((_NT, _T), jnp.int32),
            pltpu.VMEM((_T, _F), jnp.float32),
            pltpu.VMEM((_T, _F), jnp.float32),
            pltpu.VMEM((_T, _F), jnp.float32),
            pltpu.SemaphoreType.DMA,
            pltpu.SemaphoreType.DMA,
            pltpu.SemaphoreType.DMA,
            pltpu.SemaphoreType.DMA,
            pltpu.SemaphoreType.DMA,
            pltpu.SemaphoreType.DMA,
        ],
    )(_agg_body)


def _agg_body(y_hbm, src_hbm, dstl_hbm, acc_hbm,
              acc_sh, sraw, draw, src2, dst2,
              rows_a, rows_b, rows_c,
              sem_a, sem_b, sem_c,
              ssem_a, ssem_b, ssem_c):
    c = lax.axis_index("c")
    t = lax.axis_index("s")
    r0 = jnp.where(t == 0, 0, _RT0 + (t - 1) * _RT)

    for cc in range(2):
        b = c * 2 + cc
        nbase = b * _N
        e_base = b * _E + t * _EPT

        def _seed(nq, nr):
            def qb(q, carry):
                rr = r0 + q * nr
                hop = rows_a.at[pl.ds(0, nr)]
                pltpu.sync_copy(y_hbm.at[pl.ds(nbase + rr, nr)], hop)
                pltpu.sync_copy(hop, acc_sh.at[pl.ds(rr, nr)])
                return carry
            lax.fori_loop(0, nq, qb, 0)

        @pl.when(t == 0)
        def _():
            _seed(8, _RT0 // 8)

        @pl.when(t != 0)
        def _():
            _seed(13, _RT // 13)

        plsc.subcore_barrier()

        def oc_body(k, carry):
            eoff = e_base + k * _OC
            pltpu.sync_copy(src_hbm.at[pl.ds(eoff, _OC)], sraw)
            pltpu.sync_copy(dstl_hbm.at[pl.ds(eoff, _OC)], draw)
            for j in range(_NT):
                for l in range(_T // 16):
                    sl = pl.ds(l * 16, 16)
                    fl = pl.ds(j * _T + l * 16, 16)
                    src2[j, sl] = sraw[fl]
                    dst2[j, sl] = draw[fl]
            bufs = (rows_a, rows_b, rows_c)
            sems = (sem_a, sem_b, sem_c)
            ssems = (ssem_a, ssem_b, ssem_c)
            nb = 3
            gets = [None] * _NT
            puts = [None] * _NT
            gets[0] = pltpu.async_copy(y_hbm.at[src2.at[0]], bufs[0], sems[0])
            for j in range(_NT):
                if j >= nb - 1:
                    puts[j - nb + 1].wait()
                if j + 1 < _NT:
                    gets[j + 1] = pltpu.async_copy(
                        y_hbm.at[src2.at[j + 1]], bufs[(j + 1) % nb],
                        sems[(j + 1) % nb])
                gets[j].wait()
                puts[j] = pltpu.async_copy(
                    bufs[j % nb], acc_sh.at[dst2.at[j]], ssems[j % nb],
                    add=True)
            for j in range(max(0, _NT - nb + 1), _NT):
                puts[j].wait()
            return carry

        lax.fori_loop(0, _EPT // _OC, oc_body, 0)
        plsc.subcore_barrier()

        def _flush(nq, nr):
            def qb(q, carry):
                rr = r0 + q * nr
                hop = rows_a.at[pl.ds(0, nr)]
                pltpu.sync_copy(acc_sh.at[pl.ds(rr, nr)], hop)
                pltpu.sync_copy(hop, acc_hbm.at[pl.ds(nbase + rr, nr)])
                return carry
            lax.fori_loop(0, nq, qb, 0)

        @pl.when(t == 0)
        def _():
            _flush(8, _RT0 // 8)

        @pl.when(t != 0)
        def _():
            _flush(13, _RT // 13)

        plsc.subcore_barrier()


_BRM = 1000


def _mlp_body(acc_ref, x_ref, dinv_ref, bg_ref, w1_ref, b1_ref,
              w2_ref, b2_ref, out_ref, psum_ref):
    b = pl.program_id(0)
    i = pl.program_id(1)
    nb = pl.num_programs(1)
    h = acc_ref[...] * dinv_ref[...] + bg_ref[...]
    h = jnp.maximum(h, 0.0) + x_ref[...]
    h1 = jnp.dot(h, w1_ref[...], preferred_element_type=jnp.float32)
    h1 = h1 + b1_ref[...]
    h1 = jnp.where(h1 > 0, h1, 0.01 * h1)
    h2 = jnp.dot(h1, w2_ref[...], preferred_element_type=jnp.float32)
    h2 = h2 + b2_ref[...]
    h2 = jnp.where(h2 > 0, h2, 0.01 * h2)
    part = jnp.sum(h2, axis=0, keepdims=True)

    @pl.when(i == 0)
    def _():
        psum_ref[...] = part

    @pl.when(i != 0)
    def _():
        psum_ref[...] += part

    @pl.when(i == nb - 1)
    def _():
        out_ref[pl.ds(b, 1), :] = psum_ref[...]


def _mlp_stage(acc, x, dinv, bg, W1, b1, W2, b2):
    nb = _N // _BRM
    return pl.pallas_call(
        _mlp_body,
        grid=(_B, nb),
        in_specs=[
            pl.BlockSpec((_BRM, _F), lambda b, i: (b * nb + i, 0)),
            pl.BlockSpec((_BRM, _F), lambda b, i: (b * nb + i, 0)),
            pl.BlockSpec((_BRM, 1), lambda b, i: (b * nb + i, 0)),
            pl.BlockSpec((1, _F), lambda b, i: (0, 0)),
            pl.BlockSpec((_F, _H), lambda b, i: (0, 0)),
            pl.BlockSpec((1, _H), lambda b, i: (0, 0)),
            pl.BlockSpec((_H, _H), lambda b, i: (0, 0)),
            pl.BlockSpec((1, _H), lambda b, i: (0, 0)),
        ],
        out_specs=pl.BlockSpec((_B, _H), lambda b, i: (0, 0)),
        out_shape=jax.ShapeDtypeStruct((_B, _H), jnp.float32),
        scratch_shapes=[pltpu.VMEM((1, _H), jnp.float32)],
    )(acc, x, dinv, bg, W1, b1, W2, b2)


def kernel(node_features, edge_index, Wg, bg, W1, b1, W2, b2):
    x = node_features.reshape(_NN, _F)
    offs = (jnp.arange(_B, dtype=jnp.int32) * _N)[:, None]
    src_g = (edge_index[:, 0, :] + offs).reshape(_NE)
    dst_l = edge_index[:, 1, :].reshape(_NE)
    hist = _make_deg_kernel()(dst_l)
    hist_t = hist.reshape(32, _NN).T
    y, dinv = _y_stage(x, hist_t, Wg)
    acc = _make_agg_kernel()(y, src_g, dst_l)
    return _mlp_stage(acc, x, dinv, bg.reshape(1, _F), W1,
                      b1.reshape(1, _H), W2, b2.reshape(1, _H))
